# baseline (speedup 1.0000x reference)
import functools

import jax
import jax.numpy as jnp
from jax import lax
from jax.experimental import pallas as pl
from jax.experimental.pallas import tpu as pltpu
from jax.experimental.pallas import tpu_sc as plsc

V = 100000
F = 200000
B, H, W = 8, 512, 512
N = B * H * W
NC, NS, L = 2, 16, 16
NW = NC * NS
NPW = N // NW
C = 8192
NCHUNK = NPW // C
GB = 128
DEPTH = 8
RC = C // 128
RO = 3 * C // 128

_mesh = plsc.VectorSubcoreMesh(
    core_axis_name="c", subcore_axis_name="s", num_cores=NC, num_subcores=NS
)


@functools.partial(
    pl.kernel,
    out_type=jax.ShapeDtypeStruct((3 * N // 128, 128), jnp.float32),
    mesh=_mesh,
    compiler_params=pltpu.CompilerParams(
        needs_layout_passes=False, use_tc_tiling_on_sc=False
    ),
    scratch_types=[
        pltpu.VMEM((RC, 128), jnp.int32),
        pltpu.VMEM((RO, 128), jnp.float32),
        pltpu.VMEM((C,), jnp.int32),
        pltpu.VMEM((C,), jnp.int32),
        pltpu.VMEM((3 * C,), jnp.int32),
        pltpu.VMEM((RO, 128), jnp.float32),
        pltpu.SemaphoreType.DMA,
    ],
)
def _sc_shade(faces_hbm, colors_hbm, pix_hbm, bary_hbm, out_hbm,
              pixv, barf, fvidx, vid, cidx3, outf, sem):
    wid = lax.axis_index("s") * NC + lax.axis_index("c")
    iota = lax.iota(jnp.int32, L)
    iota3 = iota * 3

    for chunk in range(NCHUNK):
        base = pl.multiple_of(wid * NPW + chunk * C, C)

        pltpu.sync_copy(pix_hbm.at[pl.ds(base // 128, RC)], pixv)
        pltpu.sync_copy(bary_hbm.at[pl.ds(3 * base // 128, RO)], barf)

        def argmax_pass(t, _):
            s = pl.multiple_of(t * L, L)
            f = pixv[s >> 7, pl.ds(s & 127, L)]
            pos = 3 * s + iota3
            b0 = plsc.load_gather(barf, [pos >> 7, pos & 127])
            b1 = plsc.load_gather(barf, [(pos + 1) >> 7, (pos + 1) & 127])
            b2 = plsc.load_gather(barf, [(pos + 2) >> 7, (pos + 2) & 127])
            j = jnp.where(b1 > b0, 1, 0)
            j = jnp.where(b2 > jnp.maximum(b0, b1), 2, j)
            fvidx[pl.ds(s, L)] = f * 3 + j
            return _

        lax.fori_loop(0, C // L, argmax_pass, None)

        def gather_vid(g, _):
            o = pl.multiple_of(g * GB * DEPTH, GB)
            cps = [
                pltpu.async_copy(
                    faces_hbm.at[fvidx.at[pl.ds(o + d * GB, GB)]],
                    vid.at[pl.ds(o + d * GB, GB)],
                    sem,
                )
                for d in range(DEPTH)
            ]
            for cp in cps:
                cp.wait()
            return _

        lax.fori_loop(0, C // (GB * DEPTH), gather_vid, None)

        def expand_pass(t, _):
            s = pl.multiple_of(t * L, L)
            c = vid[pl.ds(s, L)] * 3
            pos = 3 * s + iota3
            plsc.store_scatter(cidx3, [pos], c)
            plsc.store_scatter(cidx3, [pos + 1], c + 1)
            plsc.store_scatter(cidx3, [pos + 2], c + 2)
            return _

        lax.fori_loop(0, C // L, expand_pass, None)

        def gather_color(g, _):
            ro = pl.multiple_of(g * DEPTH, 1)
            cps = [
                pltpu.async_copy(
                    colors_hbm.at[cidx3.at[pl.ds((ro + d) * GB, GB)]],
                    outf.at[ro + d],
                    sem,
                )
                for d in range(DEPTH)
            ]
            for cp in cps:
                cp.wait()
            return _

        lax.fori_loop(0, RO // DEPTH, gather_color, None)

        pltpu.sync_copy(outf, out_hbm.at[pl.ds(3 * base // 128, RO)])


def kernel(faces, verts_colors, pix_to_face, bary_coords):
    faces_flat = faces.astype(jnp.int32).reshape(3 * F)
    colors_flat = verts_colors.reshape(3 * V)
    pix = pix_to_face.astype(jnp.int32).reshape(N // 128, 128)
    bary = bary_coords.reshape(3 * N // 128, 128)
    out = _sc_shade(faces_flat, colors_flat, pix, bary)
    return out.reshape(B, H, W, 3)


# speedup vs baseline: 10.9353x; 10.9353x over previous
import functools

import jax
import jax.numpy as jnp
from jax import lax
from jax.experimental import pallas as pl
from jax.experimental.pallas import tpu as pltpu
from jax.experimental.pallas import tpu_sc as plsc

V = 100000
F = 200000
B, H, W = 8, 512, 512
N = B * H * W
NC, NS, L = 2, 16, 16
NW = NC * NS
C = 4096
NCHUNK = (N // C) // NW
GB = 128
DEPTH = 8

_mesh = plsc.VectorSubcoreMesh(
    core_axis_name="c", subcore_axis_name="s", num_cores=NC, num_subcores=NS
)


@functools.partial(
    pl.kernel,
    out_type=jax.ShapeDtypeStruct((3 * N,), jnp.float32),
    mesh=_mesh,
    compiler_params=pltpu.CompilerParams(
        needs_layout_passes=False, use_tc_tiling_on_sc=False
    ),
    scratch_types=[
        pltpu.VMEM((C,), jnp.int32),
        pltpu.VMEM((3 * C,), jnp.float32),
        pltpu.VMEM((C,), jnp.int32),
        pltpu.VMEM((C,), jnp.int32),
        pltpu.VMEM((3 * C,), jnp.int32),
        pltpu.VMEM((3 * C,), jnp.float32),
        pltpu.SemaphoreType.DMA,
    ],
)
def _sc_shade(faces_hbm, colors_hbm, pix_hbm, bary_hbm, out_hbm,
              pixv, barf, fvidx, vid, cidx, outf, sem):
    wid = lax.axis_index("s") * NC + lax.axis_index("c")
    iota = lax.iota(jnp.int32, L)

    def gather_stream(table, idx_buf, dst_buf, nidx):
        def group(g, _):
            o = pl.multiple_of(g * GB * DEPTH, GB)
            cps = [
                pltpu.async_copy(
                    table.at[idx_buf.at[pl.ds(o + d * GB, GB)]],
                    dst_buf.at[pl.ds(o + d * GB, GB)],
                    sem,
                )
                for d in range(DEPTH)
            ]
            for cp in cps:
                cp.wait()
            return _

        lax.fori_loop(0, nidx // (GB * DEPTH), group, None)

    for k in range(NCHUNK):
        g = wid * NCHUNK + k
        b = g >> 6
        hb = g & 63

        pix_off = pl.multiple_of(b * (H * W) + hb * C, C)
        bary_off = pl.multiple_of(b * (3 * H * W) + hb * (3 * C), C)
        pltpu.sync_copy(pix_hbm.at[pl.ds(pix_off, C)], pixv)
        pltpu.sync_copy(bary_hbm.at[pl.ds(bary_off, 3 * C)], barf)

        def argmax_pass(t, _):
            i = pl.multiple_of(t * L, L)
            hh = i >> 9
            w0 = i & 511
            bo = pl.multiple_of(hh * 1536 + w0, L)
            b0 = barf[pl.ds(bo, L)]
            b1 = barf[pl.ds(bo + 512, L)]
            b2 = barf[pl.ds(bo + 1024, L)]
            f = pixv[pl.ds(i, L)]
            j = jnp.where(b1 > b0, 1, 0)
            j = jnp.where(b2 > jnp.maximum(b0, b1), 2, j)
            fvidx[pl.ds(i, L)] = f * 3 + j
            return _

        lax.fori_loop(0, C // L, argmax_pass, None)

        gather_stream(faces_hbm, fvidx, vid, C)

        def expand_pass(t, _):
            i = pl.multiple_of(t * L, L)
            hh = i >> 9
            w0 = i & 511
            wt = w0 >> 7
            p0 = pl.multiple_of(wt * 1024 + hh * 128 + (w0 & 127), L)
            c0 = vid[pl.ds(i, L)] * 3
            cidx[pl.ds(p0, L)] = c0
            cidx[pl.ds(p0 + C, L)] = c0 + 1
            cidx[pl.ds(p0 + 2 * C, L)] = c0 + 2
            return _

        lax.fori_loop(0, C // L, expand_pass, None)

        gather_stream(colors_hbm, cidx, outf, 3 * C)

        out_base = (b * 3) * (H * W) + hb * C
        for c in range(3):
            pltpu.sync_copy(
                outf.at[pl.ds(c * C, C)],
                out_hbm.at[pl.ds(pl.multiple_of(out_base + c * (H * W), C), C)],
            )


def kernel(faces, verts_colors, pix_to_face, bary_coords):
    faces_flat = faces.astype(jnp.int32).reshape(3 * F)
    colors_flat = verts_colors.reshape(3 * V)
    pix = pix_to_face.astype(jnp.int32).reshape(N)
    bary = jnp.permute_dims(bary_coords, (0, 1, 3, 4, 2)).reshape(3 * N)
    out = _sc_shade(faces_flat, colors_flat, pix, bary)
    return (out.reshape(B, 3, H // 8, W // 128, 8, 128)
            .transpose(0, 2, 4, 3, 5, 1)
            .reshape(B, H, W, 3))
